# baseline (speedup 1.0000x reference)
import functools

import jax
import jax.numpy as jnp
from jax import lax
from jax.experimental import pallas as pl
from jax.experimental.pallas import tpu as pltpu
from jax.experimental.pallas import tpu_sc as plsc

_N = 10000
_E = 320000
_D = 128
_H = 128
_G = 64
_OUT = 10

_NC = 2
_NS = 16
_NW = _NC * _NS
_EPT = _E // _NW
_CHUNK = 80
_NCHUNK = _EPT // _CHUNK
_NPHASE = 5
_PCHUNK = _NCHUNK // _NPHASE
_NBUF = 4
_NPAD = 10112
_ROWS_PT = _NPAD // _NS


@functools.cache
def _make_sc_aggregate():
    mesh = plsc.VectorSubcoreMesh(core_axis_name="c", subcore_axis_name="s")

    @functools.partial(
        pl.kernel,
        out_type=jax.ShapeDtypeStruct((_NC * _NPAD, _D), jnp.float32),
        mesh=mesh,
        scratch_types=[
            pltpu.VMEM((_PCHUNK, _CHUNK), jnp.int32),
            pltpu.VMEM((_PCHUNK, _CHUNK), jnp.int32),
            pltpu.VMEM((_CHUNK, _D), jnp.float32),
            pltpu.VMEM((_CHUNK, _D), jnp.float32),
            pltpu.VMEM((_CHUNK, _D), jnp.float32),
            pltpu.VMEM((_CHUNK, _D), jnp.float32),
            pltpu.VMEM_SHARED((_NPAD, _D), jnp.float32),
            pltpu.SemaphoreType.DMA,
            pltpu.SemaphoreType.DMA,
            pltpu.SemaphoreType.DMA,
            pltpu.SemaphoreType.DMA,
            pltpu.SemaphoreType.DMA,
            pltpu.SemaphoreType.DMA,
            pltpu.SemaphoreType.DMA,
            pltpu.SemaphoreType.DMA,
        ],
    )
    def agg(x_hbm, src_hbm, dst_hbm, zeros_hbm, out_hbm,
            src_v, dst_v, rows0, rows1, rows2, rows3, acc_sh,
            gsem0, gsem1, gsem2, gsem3, ssem0, ssem1, ssem2, ssem3):
        cid = lax.axis_index("c")
        sid = lax.axis_index("s")
        wid = sid * _NC + cid

        r0 = sid * _ROWS_PT
        pltpu.sync_copy(zeros_hbm.at[pl.ds(r0, _ROWS_PT)],
                        acc_sh.at[pl.ds(r0, _ROWS_PT)])
        plsc.subcore_barrier()

        rows = (rows0, rows1, rows2, rows3)
        gsem = (gsem0, gsem1, gsem2, gsem3)
        ssem = (ssem0, ssem1, ssem2, ssem3)

        def g_start(idx, b):
            pltpu.async_copy(x_hbm.at[src_v.at[idx]], rows[b], gsem[b])

        def g_wait(idx, b):
            pltpu.make_async_copy(x_hbm.at[src_v.at[idx]], rows[b],
                                  gsem[b]).wait()

        def s_start(idx, b):
            pltpu.async_copy(rows[b], acc_sh.at[dst_v.at[idx]], ssem[b],
                             add=True)

        def s_wait(idx, b):
            pltpu.make_async_copy(rows[b], acc_sh.at[dst_v.at[idx]],
                                  ssem[b]).wait()

        def step(idx):
            b = idx % _NBUF
            g_wait(idx, b)
            s_start(idx, b)
            nx = idx + _NBUF - 1
            if nx < _PCHUNK:
                if nx >= _NBUF:
                    s_wait(nx - _NBUF, nx % _NBUF)
                g_start(nx, nx % _NBUF)

        for p in range(_NPHASE):
            slab = wid * _NPHASE + p
            pltpu.sync_copy(src_hbm.at[slab], src_v)
            pltpu.sync_copy(dst_hbm.at[slab], dst_v)

            for idx in range(_NBUF - 1):
                g_start(idx, idx)

            for idx in range(_PCHUNK):
                step(idx)
            for idx in range(_PCHUNK - _NBUF, _PCHUNK):
                s_wait(idx, idx % _NBUF)

        plsc.subcore_barrier()

        out_row = cid * _NPAD + r0
        pltpu.sync_copy(acc_sh.at[pl.ds(r0, _ROWS_PT)],
                        out_hbm.at[pl.ds(out_row, _ROWS_PT)])

    return agg


def _bn(h, gamma, beta):
    mu = jnp.mean(h, axis=0, keepdims=True)
    msq = jnp.mean(h * h, axis=0, keepdims=True)
    var = msq - mu * mu
    return gamma * (h - mu) * lax.rsqrt(var + 1e-5) + beta


def _gin_dense(x, agg2n, eps, Wa, ga, ba, Wb, gb, bb):
    agg = agg2n[0:_N, :] + agg2n[_NPAD:_NPAD + _N, :]
    h = (1.0 + eps) * x + agg
    h = jnp.dot(h, Wa, preferred_element_type=jnp.float32)
    h = _bn(h, ga, ba)
    h = jnp.maximum(h, 0.0)
    h = jnp.dot(h, Wb, preferred_element_type=jnp.float32)
    h = _bn(h, gb, bb)
    return jnp.maximum(h, 0.0)


def _tc_layer1_body(x_ref, agg_ref, eps_ref, Wa_ref, ga_ref, ba_ref,
                    Wb_ref, gb_ref, bb_ref, out_ref):
    out_ref[...] = _gin_dense(x_ref[...], agg_ref[...], eps_ref[0, 0],
                              Wa_ref[...], ga_ref[...], ba_ref[...],
                              Wb_ref[...], gb_ref[...], bb_ref[...])


_tc_layer1 = pl.pallas_call(
    _tc_layer1_body,
    out_shape=jax.ShapeDtypeStruct((_N, _H), jnp.float32),
)


def _tc_final_body(h_ref, agg_ref, batch_ref, eps_ref, Wa_ref, ga_ref, ba_ref,
                   Wb_ref, gb_ref, bb_ref, Wm1_ref, bm1_ref, Wm2_ref, bm2_ref,
                   logits_ref, probs_ref, emb_ref):
    emb = _gin_dense(h_ref[...], agg_ref[...], eps_ref[0, 0],
                     Wa_ref[...], ga_ref[...], ba_ref[...],
                     Wb_ref[...], gb_ref[...], bb_ref[...])
    emb_ref[...] = emb

    b = batch_ref[...]
    gid = lax.broadcasted_iota(jnp.int32, (_G, _N), 0)
    sel = (gid == b).astype(jnp.float32)
    sums = jnp.dot(sel, emb, preferred_element_type=jnp.float32)
    counts = jnp.sum(sel, axis=1, keepdims=True)
    pooled = sums / jnp.maximum(counts, 1.0)

    z = jnp.dot(pooled, Wm1_ref[...], preferred_element_type=jnp.float32) \
        + bm1_ref[...]
    z = jnp.where(z > 0.0, z, jnp.exp(jnp.minimum(z, 0.0)) - 1.0)
    logits = jnp.dot(z, Wm2_ref[...], preferred_element_type=jnp.float32) \
        + bm2_ref[...]
    logits_ref[...] = logits
    m = jnp.max(logits, axis=-1, keepdims=True)
    e = jnp.exp(logits - m)
    probs_ref[...] = e / jnp.sum(e, axis=-1, keepdims=True)


_tc_final = pl.pallas_call(
    _tc_final_body,
    out_shape=(
        jax.ShapeDtypeStruct((_G, _OUT), jnp.float32),
        jax.ShapeDtypeStruct((_G, _OUT), jnp.float32),
        jax.ShapeDtypeStruct((_N, _H), jnp.float32),
    ),
)


def kernel(x, edge_index, batch, eps1, W1a, g1a, b1a, W1b, g1b, b1b,
           eps2, W2a, g2a, b2a, W2b, g2b, b2b, Wm1, bm1, Wm2, bm2):
    src = edge_index[0].reshape(_NW * _NPHASE, _PCHUNK, _CHUNK)
    dst = edge_index[1].reshape(_NW * _NPHASE, _PCHUNK, _CHUNK)
    zeros = jnp.zeros((_NPAD, _D), jnp.float32)

    _sc_aggregate = _make_sc_aggregate()
    agg1 = _sc_aggregate(x, src, dst, zeros)
    h1 = _tc_layer1(x, agg1, eps1.reshape(1, 1),
                    W1a, g1a.reshape(1, -1), b1a.reshape(1, -1),
                    W1b, g1b.reshape(1, -1), b1b.reshape(1, -1))
    agg2 = _sc_aggregate(h1, src, dst, zeros)
    logits, probs, emb = _tc_final(
        h1, agg2, batch.reshape(1, -1), eps2.reshape(1, 1),
        W2a, g2a.reshape(1, -1), b2a.reshape(1, -1),
        W2b, g2b.reshape(1, -1), b2b.reshape(1, -1),
        Wm1, bm1.reshape(1, -1), Wm2, bm2.reshape(1, -1))
    return logits, probs, emb


# speedup vs baseline: 1.0215x; 1.0215x over previous
import functools

import jax
import jax.numpy as jnp
from jax import lax
from jax.experimental import pallas as pl
from jax.experimental.pallas import tpu as pltpu
from jax.experimental.pallas import tpu_sc as plsc

_N = 10000
_E = 320000
_D = 128
_H = 128
_G = 64
_OUT = 10

_NC = 2
_NS = 16
_NW = _NC * _NS
_EPT = _E // _NW
_CHUNK = 100
_NCHUNK = _EPT // _CHUNK
_NPHASE = 4
_PCHUNK = _NCHUNK // _NPHASE
_NBUF = 3
_NPAD = 10112
_ROWS_PT = _NPAD // _NS


@functools.cache
def _make_sc_aggregate():
    mesh = plsc.VectorSubcoreMesh(core_axis_name="c", subcore_axis_name="s")

    @functools.partial(
        pl.kernel,
        out_type=jax.ShapeDtypeStruct((_NC * _NPAD, _D), jnp.float32),
        mesh=mesh,
        scratch_types=[
            pltpu.VMEM((_PCHUNK, _CHUNK), jnp.int32),
            pltpu.VMEM((_PCHUNK, _CHUNK), jnp.int32),
            pltpu.VMEM((_CHUNK, _D), jnp.float32),
            pltpu.VMEM((_CHUNK, _D), jnp.float32),
            pltpu.VMEM((_CHUNK, _D), jnp.float32),
            pltpu.VMEM_SHARED((_NPAD, _D), jnp.float32),
            pltpu.SemaphoreType.DMA,
            pltpu.SemaphoreType.DMA,
            pltpu.SemaphoreType.DMA,
            pltpu.SemaphoreType.DMA,
            pltpu.SemaphoreType.DMA,
            pltpu.SemaphoreType.DMA,
        ],
    )
    def agg(x_hbm, src_hbm, dst_hbm, zeros_hbm, out_hbm,
            src_v, dst_v, rows0, rows1, rows2, acc_sh,
            gsem0, gsem1, gsem2, ssem0, ssem1, ssem2):
        cid = lax.axis_index("c")
        sid = lax.axis_index("s")
        wid = sid * _NC + cid

        r0 = sid * _ROWS_PT
        pltpu.sync_copy(zeros_hbm.at[pl.ds(r0, _ROWS_PT)],
                        acc_sh.at[pl.ds(r0, _ROWS_PT)])
        plsc.subcore_barrier()

        rows = (rows0, rows1, rows2)
        gsem = (gsem0, gsem1, gsem2)
        ssem = (ssem0, ssem1, ssem2)

        def g_start(idx, b):
            pltpu.async_copy(x_hbm.at[src_v.at[idx]], rows[b], gsem[b])

        def g_wait(idx, b):
            pltpu.make_async_copy(x_hbm.at[src_v.at[idx]], rows[b],
                                  gsem[b]).wait()

        def s_start(idx, b):
            pltpu.async_copy(rows[b], acc_sh.at[dst_v.at[idx]], ssem[b],
                             add=True)

        def s_wait(idx, b):
            pltpu.make_async_copy(rows[b], acc_sh.at[dst_v.at[idx]],
                                  ssem[b]).wait()

        def step(idx):
            b = idx % _NBUF
            g_wait(idx, b)
            s_start(idx, b)
            nx = idx + _NBUF - 1
            if nx < _PCHUNK:
                if nx >= _NBUF:
                    s_wait(nx - _NBUF, nx % _NBUF)
                g_start(nx, nx % _NBUF)

        for p in range(_NPHASE):
            slab = wid * _NPHASE + p
            pltpu.sync_copy(src_hbm.at[slab], src_v)
            pltpu.sync_copy(dst_hbm.at[slab], dst_v)

            for idx in range(_NBUF - 1):
                g_start(idx, idx)

            for idx in range(_PCHUNK):
                step(idx)
            for idx in range(_PCHUNK - _NBUF, _PCHUNK):
                s_wait(idx, idx % _NBUF)

        plsc.subcore_barrier()

        out_row = cid * _NPAD + r0
        pltpu.sync_copy(acc_sh.at[pl.ds(r0, _ROWS_PT)],
                        out_hbm.at[pl.ds(out_row, _ROWS_PT)])

    return agg


def _bn(h, gamma, beta):
    mu = jnp.mean(h, axis=0, keepdims=True)
    msq = jnp.mean(h * h, axis=0, keepdims=True)
    var = msq - mu * mu
    return gamma * (h - mu) * lax.rsqrt(var + 1e-5) + beta


def _gin_dense(x, agg2n, eps, Wa, ga, ba, Wb, gb, bb):
    agg = agg2n[0:_N, :] + agg2n[_NPAD:_NPAD + _N, :]
    h = (1.0 + eps) * x + agg
    h = jnp.dot(h, Wa, preferred_element_type=jnp.float32)
    h = _bn(h, ga, ba)
    h = jnp.maximum(h, 0.0)
    h = jnp.dot(h, Wb, preferred_element_type=jnp.float32)
    h = _bn(h, gb, bb)
    return jnp.maximum(h, 0.0)


def _tc_layer1_body(x_ref, agg_ref, eps_ref, Wa_ref, ga_ref, ba_ref,
                    Wb_ref, gb_ref, bb_ref, out_ref):
    out_ref[...] = _gin_dense(x_ref[...], agg_ref[...], eps_ref[0, 0],
                              Wa_ref[...], ga_ref[...], ba_ref[...],
                              Wb_ref[...], gb_ref[...], bb_ref[...])


_tc_layer1 = pl.pallas_call(
    _tc_layer1_body,
    out_shape=jax.ShapeDtypeStruct((_N, _H), jnp.float32),
)


def _tc_final_body(h_ref, agg_ref, batch_ref, eps_ref, Wa_ref, ga_ref, ba_ref,
                   Wb_ref, gb_ref, bb_ref, Wm1_ref, bm1_ref, Wm2_ref, bm2_ref,
                   logits_ref, probs_ref, emb_ref):
    emb = _gin_dense(h_ref[...], agg_ref[...], eps_ref[0, 0],
                     Wa_ref[...], ga_ref[...], ba_ref[...],
                     Wb_ref[...], gb_ref[...], bb_ref[...])
    emb_ref[...] = emb

    b = batch_ref[...]
    gid = lax.broadcasted_iota(jnp.int32, (_G, _N), 0)
    sel = (gid == b).astype(jnp.float32)
    sums = jnp.dot(sel, emb, preferred_element_type=jnp.float32)
    counts = jnp.sum(sel, axis=1, keepdims=True)
    pooled = sums / jnp.maximum(counts, 1.0)

    z = jnp.dot(pooled, Wm1_ref[...], preferred_element_type=jnp.float32) \
        + bm1_ref[...]
    z = jnp.where(z > 0.0, z, jnp.exp(jnp.minimum(z, 0.0)) - 1.0)
    logits = jnp.dot(z, Wm2_ref[...], preferred_element_type=jnp.float32) \
        + bm2_ref[...]
    logits_ref[...] = logits
    m = jnp.max(logits, axis=-1, keepdims=True)
    e = jnp.exp(logits - m)
    probs_ref[...] = e / jnp.sum(e, axis=-1, keepdims=True)


_tc_final = pl.pallas_call(
    _tc_final_body,
    out_shape=(
        jax.ShapeDtypeStruct((_G, _OUT), jnp.float32),
        jax.ShapeDtypeStruct((_G, _OUT), jnp.float32),
        jax.ShapeDtypeStruct((_N, _H), jnp.float32),
    ),
)


def kernel(x, edge_index, batch, eps1, W1a, g1a, b1a, W1b, g1b, b1b,
           eps2, W2a, g2a, b2a, W2b, g2b, b2b, Wm1, bm1, Wm2, bm2):
    src = edge_index[0].reshape(_NW * _NPHASE, _PCHUNK, _CHUNK)
    dst = edge_index[1].reshape(_NW * _NPHASE, _PCHUNK, _CHUNK)
    zeros = jnp.zeros((_NPAD, _D), jnp.float32)

    _sc_aggregate = _make_sc_aggregate()
    agg1 = _sc_aggregate(x, src, dst, zeros)
    h1 = _tc_layer1(x, agg1, eps1.reshape(1, 1),
                    W1a, g1a.reshape(1, -1), b1a.reshape(1, -1),
                    W1b, g1b.reshape(1, -1), b1b.reshape(1, -1))
    agg2 = _sc_aggregate(h1, src, dst, zeros)
    logits, probs, emb = _tc_final(
        h1, agg2, batch.reshape(1, -1), eps2.reshape(1, 1),
        W2a, g2a.reshape(1, -1), b2a.reshape(1, -1),
        W2b, g2b.reshape(1, -1), b2b.reshape(1, -1),
        Wm1, bm1.reshape(1, -1), Wm2, bm2.reshape(1, -1))
    return logits, probs, emb


# speedup vs baseline: 1.0926x; 1.0696x over previous
import functools

import jax
import jax.numpy as jnp
from jax import lax
from jax.experimental import pallas as pl
from jax.experimental.pallas import tpu as pltpu
from jax.experimental.pallas import tpu_sc as plsc

_N = 10000
_E = 320000
_D = 128
_H = 128
_G = 64
_OUT = 10

_NC = 2
_NS = 16
_NW = _NC * _NS
_EPT = _E // _NW
_CHUNK = 100
_NCHUNK = _EPT // _CHUNK
_NPHASE = 10
_PCHUNK = _NCHUNK // _NPHASE
_NBUF = 3
_NPAD = 10112
_ROWS_PT = _NPAD // _NS


@functools.cache
def _make_sc_aggregate():
    mesh = plsc.VectorSubcoreMesh(core_axis_name="c", subcore_axis_name="s")

    @functools.partial(
        pl.kernel,
        out_type=jax.ShapeDtypeStruct((_NC * _NPAD, _D), jnp.float32),
        mesh=mesh,
        scratch_types=[
            pltpu.VMEM((_PCHUNK, _CHUNK), jnp.int32),
            pltpu.VMEM((_PCHUNK, _CHUNK), jnp.int32),
            pltpu.VMEM((_PCHUNK, _CHUNK), jnp.int32),
            pltpu.VMEM((_PCHUNK, _CHUNK), jnp.int32),
            pltpu.VMEM((_CHUNK, _D), jnp.float32),
            pltpu.VMEM((_CHUNK, _D), jnp.float32),
            pltpu.VMEM((_CHUNK, _D), jnp.float32),
            pltpu.VMEM_SHARED((_NPAD, _D), jnp.float32),
            pltpu.SemaphoreType.DMA,
            pltpu.SemaphoreType.DMA,
            pltpu.SemaphoreType.DMA,
            pltpu.SemaphoreType.DMA,
            pltpu.SemaphoreType.DMA,
            pltpu.SemaphoreType.DMA,
            pltpu.SemaphoreType.DMA,
            pltpu.SemaphoreType.DMA,
        ],
    )
    def agg(x_hbm, src_hbm, dst_hbm, zeros_hbm, out_hbm,
            src_v0, src_v1, dst_v0, dst_v1, rows0, rows1, rows2, acc_sh,
            gsem0, gsem1, gsem2, ssem0, ssem1, ssem2, slsem_s, slsem_d):
        cid = lax.axis_index("c")
        sid = lax.axis_index("s")
        wid = sid * _NC + cid

        r0 = sid * _ROWS_PT
        pltpu.sync_copy(zeros_hbm.at[pl.ds(r0, _ROWS_PT)],
                        acc_sh.at[pl.ds(r0, _ROWS_PT)])
        plsc.subcore_barrier()

        rows = (rows0, rows1, rows2)
        gsem = (gsem0, gsem1, gsem2)
        ssem = (ssem0, ssem1, ssem2)
        svs = (src_v0, src_v1)
        dvs = (dst_v0, dst_v1)

        def g_start(k, b):
            sv = svs[(k // _PCHUNK) % 2]
            pltpu.async_copy(x_hbm.at[sv.at[k % _PCHUNK]], rows[b], gsem[b])

        def g_wait(k, b):
            sv = svs[(k // _PCHUNK) % 2]
            pltpu.make_async_copy(x_hbm.at[sv.at[k % _PCHUNK]], rows[b],
                                  gsem[b]).wait()

        def s_start(k, b):
            dv = dvs[(k // _PCHUNK) % 2]
            pltpu.async_copy(rows[b], acc_sh.at[dv.at[k % _PCHUNK]], ssem[b],
                             add=True)

        def s_wait(k, b):
            dv = dvs[(k // _PCHUNK) % 2]
            pltpu.make_async_copy(rows[b], acc_sh.at[dv.at[k % _PCHUNK]],
                                  ssem[b]).wait()

        def sl_start(p):
            slab = wid * _NPHASE + p
            pltpu.async_copy(src_hbm.at[slab], svs[p % 2], slsem_s)
            pltpu.async_copy(dst_hbm.at[slab], dvs[p % 2], slsem_d)

        def sl_wait(p):
            slab = wid * _NPHASE + p
            pltpu.make_async_copy(src_hbm.at[slab], svs[p % 2],
                                  slsem_s).wait()
            pltpu.make_async_copy(dst_hbm.at[slab], dvs[p % 2],
                                  slsem_d).wait()

        sl_start(0)
        sl_wait(0)
        sl_start(1)
        g_start(0, 0)
        g_start(1, 1)

        for k in range(_NCHUNK):
            b = k % _NBUF
            g_wait(k, b)
            s_start(k, b)
            nx = k + _NBUF - 1
            if nx < _NCHUNK:
                if nx >= _NBUF:
                    s_wait(nx - _NBUF, nx % _NBUF)
                if nx % _PCHUNK == 0 and nx >= _PCHUNK:
                    sl_wait(nx // _PCHUNK)
                g_start(nx, nx % _NBUF)
            if k >= _PCHUNK + _NBUF and (k - _NBUF) % _PCHUNK == 0:
                q = (k - _NBUF) // _PCHUNK + 1
                if q < _NPHASE:
                    sl_start(q)

        for k in range(_NCHUNK - _NBUF, _NCHUNK):
            s_wait(k, k % _NBUF)

        plsc.subcore_barrier()

        out_row = cid * _NPAD + r0
        pltpu.sync_copy(acc_sh.at[pl.ds(r0, _ROWS_PT)],
                        out_hbm.at[pl.ds(out_row, _ROWS_PT)])

    return agg


def _bn(h, gamma, beta):
    mu = jnp.mean(h, axis=0, keepdims=True)
    msq = jnp.mean(h * h, axis=0, keepdims=True)
    var = msq - mu * mu
    return gamma * (h - mu) * lax.rsqrt(var + 1e-5) + beta


def _gin_dense(x, agg2n, eps, Wa, ga, ba, Wb, gb, bb):
    agg = agg2n[0:_N, :] + agg2n[_NPAD:_NPAD + _N, :]
    h = (1.0 + eps) * x + agg
    h = jnp.dot(h, Wa, preferred_element_type=jnp.float32)
    h = _bn(h, ga, ba)
    h = jnp.maximum(h, 0.0)
    h = jnp.dot(h, Wb, preferred_element_type=jnp.float32)
    h = _bn(h, gb, bb)
    return jnp.maximum(h, 0.0)


def _tc_layer1_body(x_ref, agg_ref, eps_ref, Wa_ref, ga_ref, ba_ref,
                    Wb_ref, gb_ref, bb_ref, out_ref):
    out_ref[...] = _gin_dense(x_ref[...], agg_ref[...], eps_ref[0, 0],
                              Wa_ref[...], ga_ref[...], ba_ref[...],
                              Wb_ref[...], gb_ref[...], bb_ref[...])


_tc_layer1 = pl.pallas_call(
    _tc_layer1_body,
    out_shape=jax.ShapeDtypeStruct((_N, _H), jnp.float32),
)


def _tc_final_body(h_ref, agg_ref, batch_ref, eps_ref, Wa_ref, ga_ref, ba_ref,
                   Wb_ref, gb_ref, bb_ref, Wm1_ref, bm1_ref, Wm2_ref, bm2_ref,
                   logits_ref, probs_ref, emb_ref):
    emb = _gin_dense(h_ref[...], agg_ref[...], eps_ref[0, 0],
                     Wa_ref[...], ga_ref[...], ba_ref[...],
                     Wb_ref[...], gb_ref[...], bb_ref[...])
    emb_ref[...] = emb

    b = batch_ref[...]
    gid = lax.broadcasted_iota(jnp.int32, (_G, _N), 0)
    sel = (gid == b).astype(jnp.float32)
    sums = jnp.dot(sel, emb, preferred_element_type=jnp.float32)
    counts = jnp.sum(sel, axis=1, keepdims=True)
    pooled = sums / jnp.maximum(counts, 1.0)

    z = jnp.dot(pooled, Wm1_ref[...], preferred_element_type=jnp.float32) \
        + bm1_ref[...]
    z = jnp.where(z > 0.0, z, jnp.exp(jnp.minimum(z, 0.0)) - 1.0)
    logits = jnp.dot(z, Wm2_ref[...], preferred_element_type=jnp.float32) \
        + bm2_ref[...]
    logits_ref[...] = logits
    m = jnp.max(logits, axis=-1, keepdims=True)
    e = jnp.exp(logits - m)
    probs_ref[...] = e / jnp.sum(e, axis=-1, keepdims=True)


_tc_final = pl.pallas_call(
    _tc_final_body,
    out_shape=(
        jax.ShapeDtypeStruct((_G, _OUT), jnp.float32),
        jax.ShapeDtypeStruct((_G, _OUT), jnp.float32),
        jax.ShapeDtypeStruct((_N, _H), jnp.float32),
    ),
)


def kernel(x, edge_index, batch, eps1, W1a, g1a, b1a, W1b, g1b, b1b,
           eps2, W2a, g2a, b2a, W2b, g2b, b2b, Wm1, bm1, Wm2, bm2):
    src = edge_index[0].reshape(_NW * _NPHASE, _PCHUNK, _CHUNK)
    dst = edge_index[1].reshape(_NW * _NPHASE, _PCHUNK, _CHUNK)
    zeros = jnp.zeros((_NPAD, _D), jnp.float32)

    _sc_aggregate = _make_sc_aggregate()
    agg1 = _sc_aggregate(x, src, dst, zeros)
    h1 = _tc_layer1(x, agg1, eps1.reshape(1, 1),
                    W1a, g1a.reshape(1, -1), b1a.reshape(1, -1),
                    W1b, g1b.reshape(1, -1), b1b.reshape(1, -1))
    agg2 = _sc_aggregate(h1, src, dst, zeros)
    logits, probs, emb = _tc_final(
        h1, agg2, batch.reshape(1, -1), eps2.reshape(1, 1),
        W2a, g2a.reshape(1, -1), b2a.reshape(1, -1),
        W2b, g2b.reshape(1, -1), b2b.reshape(1, -1),
        Wm1, bm1.reshape(1, -1), Wm2, bm2.reshape(1, -1))
    return logits, probs, emb


# speedup vs baseline: 1.1068x; 1.0130x over previous
import functools

import jax
import jax.numpy as jnp
from jax import lax
from jax.experimental import pallas as pl
from jax.experimental.pallas import tpu as pltpu
from jax.experimental.pallas import tpu_sc as plsc

_N = 10000
_E = 320000
_D = 128
_H = 128
_G = 64
_OUT = 10

_NC = 2
_NS = 16
_NW = _NC * _NS
_EPT = _E // _NW
_CHUNK = 100
_NCHUNK = _EPT // _CHUNK
_NPHASE = 10
_PCHUNK = _NCHUNK // _NPHASE
_NBUF = 3
_NPAD = 10112
_ROWS_PT = _NPAD // _NS


@functools.cache
def _make_sc_aggregate():
    mesh = plsc.VectorSubcoreMesh(core_axis_name="c", subcore_axis_name="s")

    @functools.partial(
        pl.kernel,
        out_type=jax.ShapeDtypeStruct((_NC * _NPAD, _D), jnp.float32),
        mesh=mesh,
        scratch_types=[
            pltpu.VMEM((_PCHUNK, _CHUNK), jnp.int32),
            pltpu.VMEM((_PCHUNK, _CHUNK), jnp.int32),
            pltpu.VMEM((_PCHUNK, _CHUNK), jnp.int32),
            pltpu.VMEM((_PCHUNK, _CHUNK), jnp.int32),
            pltpu.VMEM((_CHUNK, _D), jnp.float32),
            pltpu.VMEM((_CHUNK, _D), jnp.float32),
            pltpu.VMEM((_CHUNK, _D), jnp.float32),
            pltpu.VMEM_SHARED((_NPAD, _D), jnp.float32),
            pltpu.SemaphoreType.DMA,
            pltpu.SemaphoreType.DMA,
            pltpu.SemaphoreType.DMA,
            pltpu.SemaphoreType.DMA,
            pltpu.SemaphoreType.DMA,
            pltpu.SemaphoreType.DMA,
            pltpu.SemaphoreType.DMA,
            pltpu.SemaphoreType.DMA,
            pltpu.SemaphoreType.DMA,
        ],
    )
    def agg(x_hbm, src_hbm, dst_hbm, zeros_hbm, out_hbm,
            src_v0, src_v1, dst_v0, dst_v1, rows0, rows1, rows2, acc_sh,
            gsem0, gsem1, gsem2, ssem0, ssem1, ssem2, slsem_s, slsem_d,
            zsem):
        cid = lax.axis_index("c")
        sid = lax.axis_index("s")
        wid = sid * _NC + cid

        r0 = sid * _ROWS_PT
        pltpu.async_copy(zeros_hbm.at[pl.ds(r0, _ROWS_PT)],
                         acc_sh.at[pl.ds(r0, _ROWS_PT)], zsem)

        rows = (rows0, rows1, rows2)
        gsem = (gsem0, gsem1, gsem2)
        ssem = (ssem0, ssem1, ssem2)
        svs = (src_v0, src_v1)
        dvs = (dst_v0, dst_v1)

        def g_start(k, b):
            sv = svs[(k // _PCHUNK) % 2]
            pltpu.async_copy(x_hbm.at[sv.at[k % _PCHUNK]], rows[b], gsem[b])

        def g_wait(k, b):
            sv = svs[(k // _PCHUNK) % 2]
            pltpu.make_async_copy(x_hbm.at[sv.at[k % _PCHUNK]], rows[b],
                                  gsem[b]).wait()

        def s_start(k, b):
            dv = dvs[(k // _PCHUNK) % 2]
            pltpu.async_copy(rows[b], acc_sh.at[dv.at[k % _PCHUNK]], ssem[b],
                             add=True)

        def s_wait(k, b):
            dv = dvs[(k // _PCHUNK) % 2]
            pltpu.make_async_copy(rows[b], acc_sh.at[dv.at[k % _PCHUNK]],
                                  ssem[b]).wait()

        def sl_start(p):
            slab = wid * _NPHASE + p
            pltpu.async_copy(src_hbm.at[slab], svs[p % 2], slsem_s)
            pltpu.async_copy(dst_hbm.at[slab], dvs[p % 2], slsem_d)

        def sl_wait(p):
            slab = wid * _NPHASE + p
            pltpu.make_async_copy(src_hbm.at[slab], svs[p % 2],
                                  slsem_s).wait()
            pltpu.make_async_copy(dst_hbm.at[slab], dvs[p % 2],
                                  slsem_d).wait()

        sl_start(0)
        sl_wait(0)
        sl_start(1)
        g_start(0, 0)
        g_start(1, 1)

        pltpu.make_async_copy(zeros_hbm.at[pl.ds(r0, _ROWS_PT)],
                              acc_sh.at[pl.ds(r0, _ROWS_PT)], zsem).wait()
        plsc.subcore_barrier()

        for k in range(_NCHUNK):
            b = k % _NBUF
            g_wait(k, b)
            s_start(k, b)
            nx = k + _NBUF - 1
            if nx < _NCHUNK:
                if nx >= _NBUF:
                    s_wait(nx - _NBUF, nx % _NBUF)
                if nx % _PCHUNK == 0 and nx >= _PCHUNK:
                    sl_wait(nx // _PCHUNK)
                g_start(nx, nx % _NBUF)
            if k >= _PCHUNK + _NBUF and (k - _NBUF) % _PCHUNK == 0:
                q = (k - _NBUF) // _PCHUNK + 1
                if q < _NPHASE:
                    sl_start(q)

        for k in range(_NCHUNK - _NBUF, _NCHUNK):
            s_wait(k, k % _NBUF)

        plsc.subcore_barrier()

        out_row = cid * _NPAD + r0
        pltpu.sync_copy(acc_sh.at[pl.ds(r0, _ROWS_PT)],
                        out_hbm.at[pl.ds(out_row, _ROWS_PT)])

    return agg


def _bn(h, gamma, beta):
    mu = jnp.mean(h, axis=0, keepdims=True)
    msq = jnp.mean(h * h, axis=0, keepdims=True)
    var = msq - mu * mu
    return gamma * (h - mu) * lax.rsqrt(var + 1e-5) + beta


def _gin_dense(x, agg2n, eps, Wa, ga, ba, Wb, gb, bb):
    agg = agg2n[0:_N, :] + agg2n[_NPAD:_NPAD + _N, :]
    h = (1.0 + eps) * x + agg
    h = jnp.dot(h, Wa, preferred_element_type=jnp.float32)
    h = _bn(h, ga, ba)
    h = jnp.maximum(h, 0.0)
    h = jnp.dot(h, Wb, preferred_element_type=jnp.float32)
    h = _bn(h, gb, bb)
    return jnp.maximum(h, 0.0)


def _tc_layer1_body(x_ref, agg_ref, eps_ref, Wa_ref, ga_ref, ba_ref,
                    Wb_ref, gb_ref, bb_ref, out_ref):
    out_ref[...] = _gin_dense(x_ref[...], agg_ref[...], eps_ref[0, 0],
                              Wa_ref[...], ga_ref[...], ba_ref[...],
                              Wb_ref[...], gb_ref[...], bb_ref[...])


_tc_layer1 = pl.pallas_call(
    _tc_layer1_body,
    out_shape=jax.ShapeDtypeStruct((_N, _H), jnp.float32),
)


def _tc_final_body(h_ref, agg_ref, batch_ref, eps_ref, Wa_ref, ga_ref, ba_ref,
                   Wb_ref, gb_ref, bb_ref, Wm1_ref, bm1_ref, Wm2_ref, bm2_ref,
                   logits_ref, probs_ref, emb_ref):
    emb = _gin_dense(h_ref[...], agg_ref[...], eps_ref[0, 0],
                     Wa_ref[...], ga_ref[...], ba_ref[...],
                     Wb_ref[...], gb_ref[...], bb_ref[...])
    emb_ref[...] = emb

    b = batch_ref[...]
    gid = lax.broadcasted_iota(jnp.int32, (_G, _N), 0)
    sel = (gid == b).astype(jnp.float32)
    sums = jnp.dot(sel, emb, preferred_element_type=jnp.float32)
    counts = jnp.sum(sel, axis=1, keepdims=True)
    pooled = sums / jnp.maximum(counts, 1.0)

    z = jnp.dot(pooled, Wm1_ref[...], preferred_element_type=jnp.float32) \
        + bm1_ref[...]
    z = jnp.where(z > 0.0, z, jnp.exp(jnp.minimum(z, 0.0)) - 1.0)
    logits = jnp.dot(z, Wm2_ref[...], preferred_element_type=jnp.float32) \
        + bm2_ref[...]
    logits_ref[...] = logits
    m = jnp.max(logits, axis=-1, keepdims=True)
    e = jnp.exp(logits - m)
    probs_ref[...] = e / jnp.sum(e, axis=-1, keepdims=True)


_tc_final = pl.pallas_call(
    _tc_final_body,
    out_shape=(
        jax.ShapeDtypeStruct((_G, _OUT), jnp.float32),
        jax.ShapeDtypeStruct((_G, _OUT), jnp.float32),
        jax.ShapeDtypeStruct((_N, _H), jnp.float32),
    ),
)


def kernel(x, edge_index, batch, eps1, W1a, g1a, b1a, W1b, g1b, b1b,
           eps2, W2a, g2a, b2a, W2b, g2b, b2b, Wm1, bm1, Wm2, bm2):
    src = edge_index[0].reshape(_NW * _NPHASE, _PCHUNK, _CHUNK)
    dst = edge_index[1].reshape(_NW * _NPHASE, _PCHUNK, _CHUNK)
    zeros = jnp.zeros((_NPAD, _D), jnp.float32)

    _sc_aggregate = _make_sc_aggregate()
    agg1 = _sc_aggregate(x, src, dst, zeros)
    h1 = _tc_layer1(x, agg1, eps1.reshape(1, 1),
                    W1a, g1a.reshape(1, -1), b1a.reshape(1, -1),
                    W1b, g1b.reshape(1, -1), b1b.reshape(1, -1))
    agg2 = _sc_aggregate(h1, src, dst, zeros)
    logits, probs, emb = _tc_final(
        h1, agg2, batch.reshape(1, -1), eps2.reshape(1, 1),
        W2a, g2a.reshape(1, -1), b2a.reshape(1, -1),
        W2b, g2b.reshape(1, -1), b2b.reshape(1, -1),
        Wm1, bm1.reshape(1, -1), Wm2, bm2.reshape(1, -1))
    return logits, probs, emb
